# baseline (device time: 120858 ns/iter reference)
import jax
import jax.numpy as jnp
from jax import lax
from jax.experimental import pallas as pl
from jax.experimental.pallas import tpu as pltpu

N_DEV = 4
B = 2
SQL = 512
H = 8
D = 64
DM = 768
HD = H * D
R = 4
G = SQL // R
SCALE = 0.125


def _perm_rows(a):
    n = a.shape[-1]
    return a.reshape(2, R, 64, n).transpose(1, 0, 2, 3).reshape(SQL, n)


def _unperm_rows(a):
    n = a.shape[-1]
    return a.reshape(R, 2, 64, n).transpose(1, 0, 2, 3).reshape(SQL, n)


def kernel(x, Wq, K_ext, V_ext, Wo):

    def body(x_ref, wq_ref, k_ref, v_ref, wo_ref, out_ref,
             kvg, send_sems, recv_sems):
        my = lax.axis_index("i")

        with jax.named_scope("stage_own"):
            for b in range(B):
                for h in range(H):
                    kvg[0, 0, b, h] = _perm_rows(
                        k_ref[b, :, h, :].astype(jnp.bfloat16))
                    kvg[0, 1, b, h] = _perm_rows(
                        v_ref[b, :, h, :].astype(jnp.bfloat16))

        with jax.named_scope("barrier"):
            barrier = pltpu.get_barrier_semaphore()
            for off in (1, 2, 3):
                pl.semaphore_signal(
                    barrier, inc=1,
                    device_id=((my + off) % N_DEV,),
                    device_id_type=pl.DeviceIdType.MESH,
                )
            pl.semaphore_wait(barrier, 3)

        with jax.named_scope("rdma_start"):
            sends = []
            for off in (1, 2, 3):
                rdma = pltpu.make_async_remote_copy(
                    src_ref=kvg.at[0],
                    dst_ref=kvg.at[N_DEV - off],
                    send_sem=send_sems.at[off - 1],
                    recv_sem=recv_sems.at[N_DEV - off],
                    device_id=((my + off) % N_DEV,),
                    device_id_type=pl.DeviceIdType.MESH,
                )
                rdma.start()
                sends.append(rdma)


        acc = [[None] * H for _ in range(B)]
        den = [[None] * H for _ in range(B)]

        def consume(s):
            for b in range(B):
                for h in range(H):
                    qh = qp[b][:, :, h * D:(h + 1) * D]
                    ks = kvg[s, 0, b, h].reshape(R, G, D)
                    vs = kvg[s, 1, b, h].reshape(R, G, D)
                    sc = lax.dot_general(
                        qh, ks, (((2,), (2,)), ((0,), (0,))),
                        preferred_element_type=jnp.float32,
                    )
                    p = jnp.exp(sc)
                    d1 = jnp.sum(p, axis=2, keepdims=True)
                    a1 = lax.dot_general(
                        p.astype(jnp.bfloat16), vs,
                        (((2,), (1,)), ((0,), (0,))),
                        preferred_element_type=jnp.float32,
                    )
                    if acc[b][h] is None:
                        acc[b][h], den[b][h] = a1, d1
                    else:
                        acc[b][h] = acc[b][h] + a1
                        den[b][h] = den[b][h] + d1


        for slot in (1, 3, 2):
            with jax.named_scope(f"wait_recv_slot{slot}"):
                recv = pltpu.make_async_remote_copy(
                    src_ref=kvg.at[0],
                    dst_ref=kvg.at[slot],
                    send_sem=send_sems.at[0],
                    recv_sem=recv_sems.at[slot],
                    device_id=(my,),
                    device_id_type=pl.DeviceIdType.MESH,
                )
                recv.wait_recv()

        with jax.named_scope("out_passthrough"):
            out_ref[...] = x_ref[...]

        with jax.named_scope("wait_send"):
            for rdma in sends:
                rdma.wait_send()

    return pl.pallas_call(
        body,
        out_shape=jax.ShapeDtypeStruct((B, SQL, DM), jnp.float32),
        in_specs=[pl.BlockSpec(memory_space=pltpu.VMEM)] * 5,
        out_specs=pl.BlockSpec(memory_space=pltpu.VMEM),
        scratch_shapes=[
            pltpu.VMEM((N_DEV, 2, B, H, SQL, D), jnp.bfloat16),
            pltpu.SemaphoreType.DMA((3,)),
            pltpu.SemaphoreType.DMA((N_DEV,)),
        ],
        compiler_params=pltpu.CompilerParams(
            collective_id=0,
            vmem_limit_bytes=100 * 1024 * 1024,
        ),
    )(x, Wq, K_ext, V_ext, Wo)


# device time: 93013 ns/iter; 1.2994x vs baseline; 1.2994x over previous
import jax
import jax.numpy as jnp
from jax import lax
from jax.experimental import pallas as pl
from jax.experimental.pallas import tpu as pltpu

N_DEV = 4
B = 2
SQL = 512
H = 8
D = 64
DM = 768
HD = H * D
R = 4
G = SQL // R
C = 2 * D
SCALE = 0.125


def _perm_rows(a):
    n = a.shape[-1]
    return a.reshape(2, R, 64, n).transpose(1, 0, 2, 3).reshape(SQL, n)


def _unperm_rows(a):
    n = a.shape[-1]
    return a.reshape(R, 2, 64, n).transpose(1, 0, 2, 3).reshape(SQL, n)


def kernel(x, Wq, K_ext, V_ext, Wo):

    def body(x_ref, wq_ref, k_ref, v_ref, wo_ref, out_ref,
             kvg, send_sems, recv_sems):
        my = lax.axis_index("i")

        with jax.named_scope("stage_own"):
            for b in range(B):
                for h in range(H):
                    kvg[0, b, h] = jnp.concatenate(
                        [
                            _perm_rows(k_ref[b, :, h, :].astype(jnp.bfloat16)),
                            _perm_rows(v_ref[b, :, h, :].astype(jnp.bfloat16)),
                        ],
                        axis=1,
                    )

        with jax.named_scope("barrier"):
            barrier = pltpu.get_barrier_semaphore()
            for off in (1, 2, 3):
                pl.semaphore_signal(
                    barrier, inc=1,
                    device_id=((my + off) % N_DEV,),
                    device_id_type=pl.DeviceIdType.MESH,
                )
            pl.semaphore_wait(barrier, 3)

        with jax.named_scope("rdma_start"):
            sends = []
            for off in (1, 2, 3):
                rdma = pltpu.make_async_remote_copy(
                    src_ref=kvg.at[0],
                    dst_ref=kvg.at[N_DEV - off],
                    send_sem=send_sems.at[off - 1],
                    recv_sem=recv_sems.at[N_DEV - off],
                    device_id=((my + off) % N_DEV,),
                    device_id_type=pl.DeviceIdType.MESH,
                )
                rdma.start()
                sends.append(rdma)

        with jax.named_scope("q_proj"):
            xb = x_ref[...].reshape(B * SQL, DM).astype(jnp.bfloat16)
            wq = wq_ref[...].astype(jnp.bfloat16)
            q = jnp.dot(xb, wq, preferred_element_type=jnp.float32)
            qp = []
            for b in range(B):
                qb = (q[b * SQL:(b + 1) * SQL] * SCALE).astype(jnp.bfloat16)
                q4 = _perm_rows(qb).reshape(R, G, H, D)
                qpad = jnp.concatenate(
                    [q4, jnp.zeros_like(q4)], axis=3
                ).reshape(R, G, H * C)
                qp.append(qpad)

        acc = [[None] * H for _ in range(B)]
        den = [[None] * H for _ in range(B)]

        def consume(s):
            for b in range(B):
                for h in range(H):
                    qh = qp[b][:, :, h * C:(h + 1) * C]
                    kvs = kvg[s, b, h].reshape(R, G, C)
                    sc = lax.dot_general(
                        qh, kvs, (((2,), (2,)), ((0,), (0,))),
                        preferred_element_type=jnp.float32,
                    )
                    p = jnp.exp(sc)
                    d1 = jnp.sum(p, axis=2, keepdims=True)
                    ce = lax.dot_general(
                        p.astype(jnp.bfloat16), kvs,
                        (((2,), (1,)), ((0,), (0,))),
                        preferred_element_type=jnp.float32,
                    )
                    a1 = ce[:, :, D:]
                    if acc[b][h] is None:
                        acc[b][h], den[b][h] = a1, d1
                    else:
                        acc[b][h] = acc[b][h] + a1
                        den[b][h] = den[b][h] + d1

        with jax.named_scope("attn_own"):
            consume(0)

        for slot in (1, 3, 2):
            with jax.named_scope(f"wait_recv_slot{slot}"):
                recv = pltpu.make_async_remote_copy(
                    src_ref=kvg.at[0],
                    dst_ref=kvg.at[slot],
                    send_sem=send_sems.at[0],
                    recv_sem=recv_sems.at[slot],
                    device_id=(my,),
                    device_id_type=pl.DeviceIdType.MESH,
                )
                recv.wait_recv()
            with jax.named_scope(f"attn_slot{slot}"):
                consume(slot)

        with jax.named_scope("out_proj"):
            wo = wo_ref[...].astype(jnp.bfloat16)
            for b in range(B):
                heads = [
                    (acc[b][h] / den[b][h]).astype(jnp.bfloat16)
                    for h in range(H)
                ]
                ctxp = jnp.concatenate(heads, axis=2).reshape(SQL, HD)
                outp = jnp.dot(ctxp, wo, preferred_element_type=jnp.float32)
                out_ref[b] = _unperm_rows(outp)

        with jax.named_scope("wait_send"):
            for rdma in sends:
                rdma.wait_send()

    return pl.pallas_call(
        body,
        out_shape=jax.ShapeDtypeStruct((B, SQL, DM), jnp.float32),
        in_specs=[pl.BlockSpec(memory_space=pltpu.VMEM)] * 5,
        out_specs=pl.BlockSpec(memory_space=pltpu.VMEM),
        scratch_shapes=[
            pltpu.VMEM((N_DEV, B, H, SQL, C), jnp.bfloat16),
            pltpu.SemaphoreType.DMA((3,)),
            pltpu.SemaphoreType.DMA((N_DEV,)),
        ],
        compiler_params=pltpu.CompilerParams(
            collective_id=0,
            vmem_limit_bytes=100 * 1024 * 1024,
        ),
    )(x, Wq, K_ext, V_ext, Wo)


# device time: 83446 ns/iter; 1.4483x vs baseline; 1.1146x over previous
import jax
import jax.numpy as jnp
from jax import lax
from jax.experimental import pallas as pl
from jax.experimental.pallas import tpu as pltpu

N_DEV = 4
B = 2
SQL = 512
H = 8
D = 64
DM = 768
HD = H * D
R = 4
G = SQL // R
C = 2 * D
SCALE = 0.125


def _perm_rows(a):
    n = a.shape[-1]
    return a.reshape(2, R, 64, n).transpose(1, 0, 2, 3).reshape(SQL, n)


def _unperm_rows(a):
    n = a.shape[-1]
    return a.reshape(R, 2, 64, n).transpose(1, 0, 2, 3).reshape(SQL, n)


def kernel(x, Wq, K_ext, V_ext, Wo):

    def body(x_ref, wq_ref, k_ref, v_ref, wo_ref, out_ref,
             kvg, send_sems, recv_sems):
        my = lax.axis_index("i")

        with jax.named_scope("barrier"):
            barrier = pltpu.get_barrier_semaphore()
            for off in (1, 2, 3):
                pl.semaphore_signal(
                    barrier, inc=1,
                    device_id=((my + off) % N_DEV,),
                    device_id_type=pl.DeviceIdType.MESH,
                )
            pl.semaphore_wait(barrier, 3)

        sends = []
        for b in range(B):
            with jax.named_scope(f"stage_own_b{b}"):
                for h in range(H):
                    kvg[0, b, h] = jnp.concatenate(
                        [
                            _perm_rows(k_ref[b, :, h, :].astype(jnp.bfloat16)),
                            _perm_rows(v_ref[b, :, h, :].astype(jnp.bfloat16)),
                        ],
                        axis=1,
                    )
            with jax.named_scope(f"rdma_start_b{b}"):
                for off in (1, 2, 3):
                    rdma = pltpu.make_async_remote_copy(
                        src_ref=kvg.at[0, b],
                        dst_ref=kvg.at[N_DEV - off, b],
                        send_sem=send_sems.at[off - 1, b],
                        recv_sem=recv_sems.at[N_DEV - off, b],
                        device_id=((my + off) % N_DEV,),
                        device_id_type=pl.DeviceIdType.MESH,
                    )
                    rdma.start()
                    sends.append(rdma)

        with jax.named_scope("q_proj"):
            xb = x_ref[...].reshape(B * SQL, DM).astype(jnp.bfloat16)
            wq = wq_ref[...].astype(jnp.bfloat16)
            q = jnp.dot(xb, wq, preferred_element_type=jnp.float32)
            qp = []
            for b in range(B):
                qb = (q[b * SQL:(b + 1) * SQL] * SCALE).astype(jnp.bfloat16)
                q4 = _perm_rows(qb).reshape(R, G, H, D)
                qpad = jnp.concatenate(
                    [q4, jnp.zeros_like(q4)], axis=3
                ).reshape(R, G, H * C)
                qp.append(qpad)

        acc = [[None] * H for _ in range(B)]
        den = [[None] * H for _ in range(B)]

        def consume_half(s, b):
            if True:
                for h in range(H):
                    qh = qp[b][:, :, h * C:(h + 1) * C]
                    kvs = kvg[s, b, h].reshape(R, G, C)
                    sc = lax.dot_general(
                        qh, kvs, (((2,), (2,)), ((0,), (0,))),
                        preferred_element_type=jnp.float32,
                    )
                    p = jnp.exp(sc)
                    d1 = jnp.sum(p, axis=2, keepdims=True)
                    ce = lax.dot_general(
                        p.astype(jnp.bfloat16), kvs,
                        (((2,), (1,)), ((0,), (0,))),
                        preferred_element_type=jnp.float32,
                    )
                    a1 = ce[:, :, D:]
                    if acc[b][h] is None:
                        acc[b][h], den[b][h] = a1, d1
                    else:
                        acc[b][h] = acc[b][h] + a1
                        den[b][h] = den[b][h] + d1

        with jax.named_scope("attn_own"):
            consume_half(0, 0)
            consume_half(0, 1)

        for slot in (1, 3, 2):
            for b in range(B):
                with jax.named_scope(f"wait_recv_s{slot}b{b}"):
                    recv = pltpu.make_async_remote_copy(
                        src_ref=kvg.at[0, b],
                        dst_ref=kvg.at[slot, b],
                        send_sem=send_sems.at[0, b],
                        recv_sem=recv_sems.at[slot, b],
                        device_id=(my,),
                        device_id_type=pl.DeviceIdType.MESH,
                    )
                    recv.wait_recv()
                with jax.named_scope(f"attn_s{slot}b{b}"):
                    consume_half(slot, b)

        with jax.named_scope("out_proj"):
            wo = wo_ref[...].astype(jnp.bfloat16)
            for b in range(B):
                heads = [
                    (acc[b][h] / den[b][h]).astype(jnp.bfloat16)
                    for h in range(H)
                ]
                ctxp = jnp.concatenate(heads, axis=2).reshape(SQL, HD)
                outp = jnp.dot(ctxp, wo, preferred_element_type=jnp.float32)
                out_ref[b] = _unperm_rows(outp)

        with jax.named_scope("wait_send"):
            for rdma in sends:
                rdma.wait_send()

    return pl.pallas_call(
        body,
        out_shape=jax.ShapeDtypeStruct((B, SQL, DM), jnp.float32),
        in_specs=[pl.BlockSpec(memory_space=pltpu.VMEM)] * 5,
        out_specs=pl.BlockSpec(memory_space=pltpu.VMEM),
        scratch_shapes=[
            pltpu.VMEM((N_DEV, B, H, SQL, C), jnp.bfloat16),
            pltpu.SemaphoreType.DMA((3, B)),
            pltpu.SemaphoreType.DMA((N_DEV, B)),
        ],
        compiler_params=pltpu.CompilerParams(
            collective_id=0,
            vmem_limit_bytes=100 * 1024 * 1024,
        ),
    )(x, Wq, K_ext, V_ext, Wo)


# device time: 78918 ns/iter; 1.5314x vs baseline; 1.0574x over previous
import jax
import jax.numpy as jnp
from jax import lax
from jax.experimental import pallas as pl
from jax.experimental.pallas import tpu as pltpu

N_DEV = 4
B = 2
SQL = 512
H = 8
D = 64
DM = 768
HD = H * D
R = 4
G = SQL // R
C = 2 * D
SCALE = 0.125


def _perm_rows(a):
    n = a.shape[-1]
    return a.reshape(2, R, 64, n).transpose(1, 0, 2, 3).reshape(SQL, n)


def _unperm_rows(a):
    n = a.shape[-1]
    return a.reshape(R, 2, 64, n).transpose(1, 0, 2, 3).reshape(SQL, n)


def kernel(x, Wq, K_ext, V_ext, Wo):

    def body(x_ref, wq_ref, k_ref, v_ref, wo_ref, out_ref,
             kvg, send_sems, recv_sems):
        my = lax.axis_index("i")

        with jax.named_scope("barrier"):
            barrier = pltpu.get_barrier_semaphore()
            for off in (1, 2, 3):
                pl.semaphore_signal(
                    barrier, inc=1,
                    device_id=((my + off) % N_DEV,),
                    device_id_type=pl.DeviceIdType.MESH,
                )
            pl.semaphore_wait(barrier, 3)

        sends = []
        for b in range(B):
            with jax.named_scope(f"stage_own_b{b}"):
                for h in range(H):
                    kvg[0, b, h] = jnp.concatenate(
                        [
                            _perm_rows(k_ref[b, :, h, :].astype(jnp.bfloat16)),
                            _perm_rows(v_ref[b, :, h, :].astype(jnp.bfloat16)),
                        ],
                        axis=1,
                    )
            with jax.named_scope(f"rdma_start_b{b}"):
                for off in (1, 2, 3):
                    rdma = pltpu.make_async_remote_copy(
                        src_ref=kvg.at[0, b],
                        dst_ref=kvg.at[N_DEV - off, b],
                        send_sem=send_sems.at[off - 1, b],
                        recv_sem=recv_sems.at[N_DEV - off, b],
                        device_id=((my + off) % N_DEV,),
                        device_id_type=pl.DeviceIdType.MESH,
                    )
                    rdma.start()
                    sends.append(rdma)

        with jax.named_scope("q_proj"):
            xb = x_ref[...].reshape(B * SQL, DM).astype(jnp.bfloat16)
            wq = wq_ref[...].astype(jnp.bfloat16)
            q = jnp.dot(xb, wq, preferred_element_type=jnp.float32)
            qp = []
            for b in range(B):
                qb = (q[b * SQL:(b + 1) * SQL] * SCALE).astype(jnp.bfloat16)
                q4 = _perm_rows(qb).reshape(R, G, H, D)
                qpad = jnp.concatenate(
                    [q4, jnp.zeros_like(q4)], axis=3
                ).reshape(R, G, H * C)
                qp.append(qpad)

        acc = [[None] * H for _ in range(B)]
        den = [[None] * H for _ in range(B)]

        def consume_half(s, b):
            if True:
                for h in range(H):
                    qh = qp[b][:, :, h * C:(h + 1) * C]
                    kvs = kvg[s, b, h].reshape(R, G, C)
                    sc = lax.dot_general(
                        qh, kvs, (((2,), (2,)), ((0,), (0,))),
                        preferred_element_type=jnp.float32,
                    )
                    p = jnp.exp(sc.astype(jnp.bfloat16))
                    d1 = jnp.sum(p.astype(jnp.float32), axis=2,
                                 keepdims=True)
                    ce = lax.dot_general(
                        p, kvs,
                        (((2,), (1,)), ((0,), (0,))),
                        preferred_element_type=jnp.float32,
                    )
                    a1 = ce[:, :, D:]
                    if acc[b][h] is None:
                        acc[b][h], den[b][h] = a1, d1
                    else:
                        acc[b][h] = acc[b][h] + a1
                        den[b][h] = den[b][h] + d1

        with jax.named_scope("attn_own"):
            consume_half(0, 0)
            consume_half(0, 1)

        wo = wo_ref[...].astype(jnp.bfloat16)

        def project_out(b):
            heads = [
                (acc[b][h] / den[b][h]).astype(jnp.bfloat16)
                for h in range(H)
            ]
            ctxp = jnp.concatenate(heads, axis=2).reshape(SQL, HD)
            outp = jnp.dot(ctxp, wo, preferred_element_type=jnp.float32)
            out_ref[b] = _unperm_rows(outp)

        for b in range(B):
            for slot in (1, 3, 2):
                with jax.named_scope(f"wait_recv_s{slot}b{b}"):
                    recv = pltpu.make_async_remote_copy(
                        src_ref=kvg.at[0, b],
                        dst_ref=kvg.at[slot, b],
                        send_sem=send_sems.at[0, b],
                        recv_sem=recv_sems.at[slot, b],
                        device_id=(my,),
                        device_id_type=pl.DeviceIdType.MESH,
                    )
                    recv.wait_recv()
                with jax.named_scope(f"attn_s{slot}b{b}"):
                    consume_half(slot, b)
            with jax.named_scope(f"out_proj_b{b}"):
                project_out(b)

        with jax.named_scope("wait_send"):
            for rdma in sends:
                rdma.wait_send()

    return pl.pallas_call(
        body,
        out_shape=jax.ShapeDtypeStruct((B, SQL, DM), jnp.float32),
        in_specs=[pl.BlockSpec(memory_space=pltpu.VMEM)] * 5,
        out_specs=pl.BlockSpec(memory_space=pltpu.VMEM),
        scratch_shapes=[
            pltpu.VMEM((N_DEV, B, H, SQL, C), jnp.bfloat16),
            pltpu.SemaphoreType.DMA((3, B)),
            pltpu.SemaphoreType.DMA((N_DEV, B)),
        ],
        compiler_params=pltpu.CompilerParams(
            collective_id=0,
            vmem_limit_bytes=100 * 1024 * 1024,
        ),
    )(x, Wq, K_ext, V_ext, Wo)
